# baseline (speedup 1.0000x reference)
import functools

import jax
import jax.numpy as jnp
from jax.experimental import pallas as pl
from jax.experimental.pallas import tpu as pltpu

_NBUF_IN = 6
_NBUF_OUT = 4


def _pick_chunk(n):
    for c in (800, 200, 8):
        if n % c == 0:
            return c
    return n


def _ring_kernel(a_hbm, g_hbm, o_hbm, a_buf, g_buf, o_buf, ids, macc, iacc,
                 a_sem, g_sem, o_sem, *, n, m, chunk):
    nchunks = n // chunk

    def a_copy(c, s):
        return pltpu.make_async_copy(
            a_hbm.at[pl.ds(c * chunk, chunk), :], a_buf.at[s], a_sem.at[s])

    def g_copy(c, s):
        return pltpu.make_async_copy(
            g_hbm.at[pl.ds(c * chunk, chunk), :], g_buf.at[s], g_sem.at[s])

    def o_copy(c, s):
        return pltpu.make_async_copy(
            o_buf.at[s], o_hbm.at[pl.ds(c * chunk, chunk), :], o_sem.at[s])

    ids[...] = jax.lax.broadcasted_iota(jnp.int32, (chunk, m), 0)
    macc[...] = jnp.full((1, m), -jnp.inf, jnp.float32)
    iacc[...] = jnp.zeros((1, m), jnp.int32)

    for s in range(min(_NBUF_IN, nchunks)):
        a_copy(s, s).start()
        g_copy(s, s).start()

    def body_a(i, carry):
        s = jax.lax.rem(i, _NBUF_IN)
        a_copy(i, s).wait()
        g_copy(i, s).wait()

        x = a_buf[s] + g_buf[s]
        bm = jnp.max(x, axis=0, keepdims=True)
        bi = jnp.min(jnp.where(x >= bm, ids[...], jnp.int32(n)), axis=0,
                     keepdims=True) + i * chunk
        better = bm > macc[...]
        iacc[...] = jnp.where(better, bi, iacc[...])
        macc[...] = jnp.maximum(bm, macc[...])

        @pl.when(i + _NBUF_IN < nchunks)
        def _():
            a_copy(i + _NBUF_IN, s).start()
            g_copy(i + _NBUF_IN, s).start()

        return carry

    jax.lax.fori_loop(0, nchunks, body_a, 0)

    def body_b(i, carry):
        s = jax.lax.rem(i, _NBUF_OUT)

        @pl.when(i >= _NBUF_OUT)
        def _():
            o_copy(i - _NBUF_OUT, s).wait()

        rel = iacc[...] - i * chunk
        o_buf[s] = (ids[...] == rel).astype(jnp.float32)
        o_copy(i, s).start()
        return carry

    jax.lax.fori_loop(0, nchunks, body_b, 0)
    for c in range(max(nchunks - _NBUF_OUT, 0), nchunks):
        o_copy(c, c % _NBUF_OUT).wait()


def kernel(alpha, gumbel, tau):
    del tau
    mm, kk = alpha.shape
    n, m = kk, mm
    chunk = _pick_chunk(n)
    inbuf = lambda: pltpu.VMEM((_NBUF_IN, chunk, m), jnp.float32)
    outbuf = pltpu.VMEM((_NBUF_OUT, chunk, m), jnp.float32)
    out_t = pl.pallas_call(
        functools.partial(_ring_kernel, n=n, m=m, chunk=chunk),
        in_specs=[
            pl.BlockSpec(memory_space=pl.ANY),
            pl.BlockSpec(memory_space=pl.ANY),
        ],
        out_specs=pl.BlockSpec(memory_space=pl.ANY),
        out_shape=jax.ShapeDtypeStruct((n, m), jnp.float32),
        scratch_shapes=[
            inbuf(), inbuf(), outbuf,
            pltpu.VMEM((chunk, m), jnp.int32),
            pltpu.VMEM((1, m), jnp.float32),
            pltpu.VMEM((1, m), jnp.int32),
            pltpu.SemaphoreType.DMA((_NBUF_IN,)),
            pltpu.SemaphoreType.DMA((_NBUF_IN,)),
            pltpu.SemaphoreType.DMA((_NBUF_OUT,)),
        ],
    )(alpha.T, gumbel.T)
    return out_t.T
